# Initial kernel scaffold; baseline (speedup 1.0000x reference)
import jax
import jax.numpy as jnp
from jax.experimental import pallas as pl


def kernel(x, norm, edge_index, edge_type, rFeatures, relation_weights, loop_weight):
    raise NotImplementedError("write your pallas kernel here")


# speedup vs baseline: 4.1013x; 4.1013x over previous
import functools

import jax
import jax.numpy as jnp
from jax import lax
from jax.experimental import pallas as pl
from jax.experimental.pallas import tpu as pltpu
from jax.experimental.pallas import tpu_sc as plsc

N = 10000
E = 160000
D = 256
R = 16
H = 128
NR = N * R
NSUB = 16
EP = E // NSUB
G = 80
CH = EP // G
BN = 1000
NB = N // BN


def _transform_body(apply_scale, h2_ref, scale_ref, wp_ref, rfp_ref, out_ref):
    h = jnp.concatenate([h2_ref[0], h2_ref[1]], axis=-1)
    if apply_scale:
        h = h * scale_ref[...]
    p = jnp.dot(h, wp_ref[0], preferred_element_type=jnp.float32)
    p = p.reshape(BN, R, H) + rfp_ref[0][None]
    out_ref[0] = p


def _transform(h2, scale, wp, rfp, apply_scale):
    return pl.pallas_call(
        functools.partial(_transform_body, apply_scale),
        grid=(NB, 2),
        in_specs=[
            pl.BlockSpec((2, BN, H), lambda i, c: (0, i, 0)),
            pl.BlockSpec((BN, 1), lambda i, c: (i, 0)),
            pl.BlockSpec((1, D, R * H), lambda i, c: (c, 0, 0)),
            pl.BlockSpec((1, R, H), lambda i, c: (c, 0, 0)),
        ],
        out_specs=pl.BlockSpec((1, BN, R, H), lambda i, c: (c, i, 0, 0)),
        out_shape=jax.ShapeDtypeStruct((2, N, R, H), jnp.float32),
    )(h2, scale, wp, rfp)


@functools.cache
def _make_sc_scatter():
    mesh = plsc.VectorSubcoreMesh(core_axis_name="c", subcore_axis_name="s")
    return functools.partial(
        pl.kernel,
        mesh=mesh,
        out_type=jax.ShapeDtypeStruct((2, N, H), jnp.float32),
        scratch_types=[
            pltpu.VMEM((EP,), jnp.int32),
            pltpu.VMEM((EP,), jnp.int32),
            pltpu.VMEM((G, H), jnp.float32),
            pltpu.VMEM((G,), jnp.int32),
            pltpu.VMEM((G,), jnp.int32),
            pltpu.VMEM_SHARED((N, H), jnp.float32),
            pltpu.SemaphoreType.DMA,
        ],
    )(_sc_scatter_body)


def _sc_scatter_body(key_hbm, dst_hbm, t_hbm, out_hbm,
                     key_v, dst_v, rows_v, keybuf, dstbuf, acc, sem):
    c = lax.axis_index("c")
    s = lax.axis_index("s")
    base = s * EP
    pltpu.sync_copy(key_hbm.at[pl.ds(base, EP)], key_v)
    pltpu.sync_copy(dst_hbm.at[pl.ds(base, EP)], dst_v)

    def _zero(i, carry):
        r = i // (H // 16)
        col = (i % (H // 16)) * 16
        rows_v[r, pl.ds(col, 16)] = jnp.zeros((16,), jnp.float32)
        return carry
    lax.fori_loop(0, G * (H // 16), _zero, 0)

    zbase = s * (N // NSUB)
    for k in range(7):
        pltpu.sync_copy(rows_v, acc.at[pl.ds(zbase + k * G, G)])
    pltpu.sync_copy(rows_v.at[pl.ds(0, 65)], acc.at[pl.ds(zbase + 7 * G, 65)])
    plsc.subcore_barrier()

    koff = c * NR

    def _chunk(k, carry):
        cb = k * G
        def _stage(j, inner):
            sl = pl.ds(j * 16, 16)
            keybuf[sl] = key_v[pl.ds(cb + j * 16, 16)] + koff
            dstbuf[sl] = dst_v[pl.ds(cb + j * 16, 16)]
            return inner
        lax.fori_loop(0, G // 16, _stage, 0)
        pltpu.async_copy(t_hbm.at[keybuf], rows_v, sem).wait()
        pltpu.sync_copy(rows_v, acc.at[dstbuf], add=True)
        return carry
    lax.fori_loop(0, CH, _chunk, 0)
    plsc.subcore_barrier()

    rbase = s * 624
    pltpu.sync_copy(acc.at[pl.ds(rbase, 624)], out_hbm.at[c, pl.ds(rbase, 624)])
    @pl.when(s == NSUB - 1)
    def _tail():
        pltpu.sync_copy(acc.at[pl.ds(9984, 16)], out_hbm.at[c, pl.ds(9984, 16)])


def _final_body(agg_ref, norm_ref, lw_ref, out_ref):
    h2 = jnp.concatenate([agg_ref[0], agg_ref[1]], axis=-1)
    h2 = h2 * norm_ref[...]
    out_ref[...] = jnp.maximum(
        h2 + jnp.dot(h2, lw_ref[...], preferred_element_type=jnp.float32), 0.0)


def _final(agg, normv, loop_weight):
    return pl.pallas_call(
        _final_body,
        grid=(NB,),
        in_specs=[
            pl.BlockSpec((2, BN, H), lambda i: (0, i, 0)),
            pl.BlockSpec((BN, 1), lambda i: (i, 0)),
            pl.BlockSpec((D, D), lambda i: (0, 0)),
        ],
        out_specs=pl.BlockSpec((BN, D), lambda i: (i, 0)),
        out_shape=jax.ShapeDtypeStruct((N, D), jnp.float32),
    )(agg, normv, loop_weight)


def kernel(x, norm, edge_index, edge_type, rFeatures, relation_weights, loop_weight):
    src = edge_index[0].astype(jnp.int32)
    dst = edge_index[1].astype(jnp.int32)
    key = src * R + edge_type.astype(jnp.int32)

    wp = relation_weights.reshape(R, D, 2, H).transpose(2, 1, 0, 3).reshape(2, D, R * H)
    rfp = rFeatures.reshape(R, 2, H).transpose(1, 0, 2)
    normv = norm.reshape(N, 1)
    x2 = x.reshape(N, 2, H).transpose(1, 0, 2)

    sc_scatter = _make_sc_scatter()
    t0 = _transform(x2, normv, wp, rfp, apply_scale=False)
    agg1 = sc_scatter(key, dst, t0.reshape(2 * NR, H))
    t1 = _transform(agg1, normv, wp, rfp, apply_scale=True)
    agg2 = sc_scatter(key, dst, t1.reshape(2 * NR, H))
    return _final(agg2, normv, loop_weight)
